# baseline (device time: 2934942 ns/iter reference)
import jax
import jax.numpy as jnp
from jax import lax
from jax.experimental import pallas as pl
from jax.experimental.pallas import tpu as pltpu

N_DEV = 32


def kernel(x, w_mat):
    m_per, k = x.shape
    _, n_per = w_mat.shape

    def _gemm(chunk, w):
        y = jnp.dot(chunk, w, preferred_element_type=jnp.float32)
        return jnp.maximum(y, 0.0)

    def body(x_ref, w_ref, out_ref, comm_ref, send_sems, recv_sems):
        my_pos = lax.axis_index("i")
        left = lax.rem(my_pos - 1 + N_DEV, N_DEV)
        right = lax.rem(my_pos + 1, N_DEV)

        barrier_sem = pltpu.get_barrier_semaphore()
        for nbr in (left, right):
            pl.semaphore_signal(
                barrier_sem, inc=1,
                device_id=(nbr,), device_id_type=pl.DeviceIdType.MESH,
            )
        pl.semaphore_wait(barrier_sem, 2)

        comm_ref[0] = x_ref[...]
        out_ref[pl.ds(my_pos * m_per, m_per), :] = _gemm(x_ref[...], w_ref[...])

        for h in range(N_DEV - 1):
            send_slot = h % 2
            recv_slot = (h + 1) % 2
            rdma = pltpu.make_async_remote_copy(
                src_ref=comm_ref.at[send_slot],
                dst_ref=comm_ref.at[recv_slot],
                send_sem=send_sems.at[send_slot],
                recv_sem=recv_sems.at[recv_slot],
                device_id=(right,),
                device_id_type=pl.DeviceIdType.MESH,
            )
            rdma.start()
            rdma.wait()

            origin = lax.rem(my_pos - h - 1 + N_DEV, N_DEV)
            out_ref[pl.ds(origin * m_per, m_per), :] = _gemm(
                comm_ref[recv_slot], w_ref[...]
            )

    return pl.pallas_call(
        body,
        out_shape=jax.ShapeDtypeStruct((N_DEV * m_per, n_per), jnp.float32),
        in_specs=[
            pl.BlockSpec(memory_space=pltpu.VMEM),
            pl.BlockSpec(memory_space=pltpu.VMEM),
        ],
        out_specs=pl.BlockSpec(memory_space=pltpu.VMEM),
        scratch_shapes=[
            pltpu.VMEM((2, m_per, k), jnp.float32),
            pltpu.SemaphoreType.DMA((2,)),
            pltpu.SemaphoreType.DMA((2,)),
        ],
        compiler_params=pltpu.CompilerParams(collective_id=0),
    )(x, w_mat)


# device time: 1490253 ns/iter; 1.9694x vs baseline; 1.9694x over previous
import jax
import jax.numpy as jnp
from jax import lax
from jax.experimental import pallas as pl
from jax.experimental.pallas import tpu as pltpu

N_DEV = 32
N_STEPS = 16

_P = [(0, 0), (1, 0), (2, 0), (3, 0), (3, 1), (2, 1), (1, 1), (0, 1),
      (0, 2), (1, 2), (2, 2), (3, 2), (3, 3), (2, 3), (1, 3), (0, 3)]
_CYCLE_COORDS = [(0, y, z) for y, z in _P] + [(1, y, z) for y, z in reversed(_P)]


def _logical(x, y, z):
    return z * 8 + y * 2 + (x if y % 2 == 0 else 1 - x)


_CYCLE = [_logical(*c) for c in _CYCLE_COORDS]
assert sorted(_CYCLE) == list(range(N_DEV))
_INV = [0] * N_DEV
for _i, _p in enumerate(_CYCLE):
    _INV[_p] = _i


def kernel(x, w_mat):
    m_per, k = x.shape
    _, n_per = w_mat.shape

    my_pos = lax.axis_index("i")
    cyc = jnp.asarray(_CYCLE, jnp.int32)
    inv = jnp.asarray(_INV, jnp.int32)
    cpos = inv[my_pos]
    steps = jnp.arange(N_STEPS + 1, dtype=jnp.int32)
    nbrs = jnp.stack([cyc[(cpos - 1) % N_DEV], cyc[(cpos + 1) % N_DEV]])
    origin_r = cyc[(cpos - steps) % N_DEV]
    origin_l = cyc[(cpos + steps[:N_STEPS]) % N_DEV]

    def _gemm(chunk, w):
        y = jnp.dot(chunk, w, preferred_element_type=jnp.float32)
        return jnp.maximum(y, 0.0)

    def body(nbr_ref, or_ref, ol_ref, x_ref, w_ref, out_ref,
             rbuf, lbuf, r_send, r_recv, l_send, l_recv, cred_r, cred_l):
        cl = nbr_ref[0]
        cr = nbr_ref[1]

        barrier = pltpu.get_barrier_semaphore()
        for nbr in (cl, cr):
            pl.semaphore_signal(
                barrier, inc=1,
                device_id=(nbr,), device_id_type=pl.DeviceIdType.MESH,
            )
        pl.semaphore_wait(barrier, 2)

        rbuf[0] = x_ref[...]
        lbuf[0] = x_ref[...]

        for t in range(N_STEPS):
            s = t % 2
            r = (t + 1) % 2
            if 1 <= t:
                pl.semaphore_wait(cred_r, 1)
            if 1 <= t <= N_STEPS - 2:
                pl.semaphore_wait(cred_l, 1)

            r_rdma = pltpu.make_async_remote_copy(
                src_ref=rbuf.at[s], dst_ref=rbuf.at[r],
                send_sem=r_send.at[s], recv_sem=r_recv.at[r],
                device_id=(cr,), device_id_type=pl.DeviceIdType.MESH,
            )
            r_rdma.start()
            if t <= N_STEPS - 2:
                l_rdma = pltpu.make_async_remote_copy(
                    src_ref=lbuf.at[s], dst_ref=lbuf.at[r],
                    send_sem=l_send.at[s], recv_sem=l_recv.at[r],
                    device_id=(cl,), device_id_type=pl.DeviceIdType.MESH,
                )
                l_rdma.start()

            if t == 0:
                out_ref[pl.ds(or_ref[0] * m_per, m_per), :] = _gemm(
                    x_ref[...], w_ref[...])
            else:
                out_ref[pl.ds(or_ref[t] * m_per, m_per), :] = _gemm(
                    rbuf[s], w_ref[...])
                out_ref[pl.ds(ol_ref[t] * m_per, m_per), :] = _gemm(
                    lbuf[s], w_ref[...])

            r_rdma.wait_send()
            if t <= N_STEPS - 2:
                l_rdma.wait_send()
            if t <= N_STEPS - 2:
                pl.semaphore_signal(
                    cred_r, inc=1,
                    device_id=(cl,), device_id_type=pl.DeviceIdType.MESH,
                )
            if t <= N_STEPS - 3:
                pl.semaphore_signal(
                    cred_l, inc=1,
                    device_id=(cr,), device_id_type=pl.DeviceIdType.MESH,
                )

            r_rdma.wait_recv()
            if t <= N_STEPS - 2:
                l_rdma.wait_recv()

        out_ref[pl.ds(or_ref[N_STEPS] * m_per, m_per), :] = _gemm(
            rbuf[0], w_ref[...])

    return pl.pallas_call(
        body,
        out_shape=jax.ShapeDtypeStruct((N_DEV * m_per, n_per), jnp.float32),
        in_specs=[
            pl.BlockSpec(memory_space=pltpu.SMEM),
            pl.BlockSpec(memory_space=pltpu.SMEM),
            pl.BlockSpec(memory_space=pltpu.SMEM),
            pl.BlockSpec(memory_space=pltpu.VMEM),
            pl.BlockSpec(memory_space=pltpu.VMEM),
        ],
        out_specs=pl.BlockSpec(memory_space=pltpu.VMEM),
        scratch_shapes=[
            pltpu.VMEM((2, m_per, k), jnp.float32),
            pltpu.VMEM((2, m_per, k), jnp.float32),
            pltpu.SemaphoreType.DMA((2,)),
            pltpu.SemaphoreType.DMA((2,)),
            pltpu.SemaphoreType.DMA((2,)),
            pltpu.SemaphoreType.DMA((2,)),
            pltpu.SemaphoreType.REGULAR,
            pltpu.SemaphoreType.REGULAR,
        ],
        compiler_params=pltpu.CompilerParams(collective_id=0),
    )(nbrs, origin_r, origin_l, x, w_mat)


# device time: 1445133 ns/iter; 2.0309x vs baseline; 1.0312x over previous
import jax
import jax.numpy as jnp
from jax import lax
from jax.experimental import pallas as pl
from jax.experimental.pallas import tpu as pltpu

N_DEV = 32
N_STEPS = 16

_P = [(0, 0), (1, 0), (2, 0), (3, 0), (3, 1), (2, 1), (1, 1), (0, 1),
      (0, 2), (1, 2), (2, 2), (3, 2), (3, 3), (2, 3), (1, 3), (0, 3)]
_CYCLE_COORDS = [(0, y, z) for y, z in _P] + [(1, y, z) for y, z in reversed(_P)]


def _logical(x, y, z):
    return z * 8 + y * 2 + (x if y % 2 == 0 else 1 - x)


_CYCLE = [_logical(*c) for c in _CYCLE_COORDS]
assert sorted(_CYCLE) == list(range(N_DEV))
_INV = [0] * N_DEV
for _i, _p in enumerate(_CYCLE):
    _INV[_p] = _i


def kernel(x, w_mat):
    m_per, k = x.shape
    _, n_per = w_mat.shape

    my_pos = lax.axis_index("i")
    cyc = jnp.asarray(_CYCLE, jnp.int32)
    inv = jnp.asarray(_INV, jnp.int32)
    cpos = inv[my_pos]
    steps = jnp.arange(N_STEPS + 1, dtype=jnp.int32)
    nbrs = jnp.stack([cyc[(cpos - 1) % N_DEV], cyc[(cpos + 1) % N_DEV]])
    origin_r = cyc[(cpos - steps) % N_DEV]
    origin_l = cyc[(cpos + steps[:N_STEPS]) % N_DEV]

    def _gemm(chunk, w):
        y = jnp.dot(chunk, w, preferred_element_type=jnp.float32)
        return jnp.maximum(y, 0.0)

    def body(nbr_ref, or_ref, ol_ref, x_ref, w_ref, out_ref,
             rbuf, lbuf, r_send, r_recv, l_send, l_recv, cred_r, cred_l):
        cl = nbr_ref[0]
        cr = nbr_ref[1]

        barrier = pltpu.get_barrier_semaphore()
        for nbr in (cl, cr):
            pl.semaphore_signal(
                barrier, inc=1,
                device_id=(nbr,), device_id_type=pl.DeviceIdType.MESH,
            )
        pl.semaphore_wait(barrier, 2)

        rbuf[0] = x_ref[...]
        lbuf[0] = x_ref[...]

        for t in range(N_STEPS):
            s = t % 2
            r = (t + 1) % 2
            if 1 <= t:
                pl.semaphore_wait(cred_r, 1)
                pl.semaphore_wait(cred_l, 1)

            if t <= N_STEPS - 2:
                r_rdma = pltpu.make_async_remote_copy(
                    src_ref=rbuf.at[s], dst_ref=rbuf.at[r],
                    send_sem=r_send.at[s], recv_sem=r_recv.at[r],
                    device_id=(cr,), device_id_type=pl.DeviceIdType.MESH,
                )
                l_rdma = pltpu.make_async_remote_copy(
                    src_ref=lbuf.at[s], dst_ref=lbuf.at[r],
                    send_sem=l_send.at[s], recv_sem=l_recv.at[r],
                    device_id=(cl,), device_id_type=pl.DeviceIdType.MESH,
                )
            else:
                h = m_per // 2
                r_rdma = pltpu.make_async_remote_copy(
                    src_ref=rbuf.at[s, pl.ds(0, h)],
                    dst_ref=rbuf.at[r, pl.ds(0, h)],
                    send_sem=r_send.at[s], recv_sem=r_recv.at[r],
                    device_id=(cr,), device_id_type=pl.DeviceIdType.MESH,
                )
                l_rdma = pltpu.make_async_remote_copy(
                    src_ref=lbuf.at[s, pl.ds(h, h)],
                    dst_ref=lbuf.at[r, pl.ds(h, h)],
                    send_sem=l_send.at[s], recv_sem=l_recv.at[r],
                    device_id=(cl,), device_id_type=pl.DeviceIdType.MESH,
                )
            r_rdma.start()
            l_rdma.start()

            if t == 0:
                out_ref[pl.ds(or_ref[0] * m_per, m_per), :] = _gemm(
                    x_ref[...], w_ref[...])
            else:
                out_ref[pl.ds(or_ref[t] * m_per, m_per), :] = _gemm(
                    rbuf[s], w_ref[...])
                out_ref[pl.ds(ol_ref[t] * m_per, m_per), :] = _gemm(
                    lbuf[s], w_ref[...])

            r_rdma.wait_send()
            l_rdma.wait_send()
            if t <= N_STEPS - 2:
                pl.semaphore_signal(
                    cred_r, inc=1,
                    device_id=(cl,), device_id_type=pl.DeviceIdType.MESH,
                )
                pl.semaphore_signal(
                    cred_l, inc=1,
                    device_id=(cr,), device_id_type=pl.DeviceIdType.MESH,
                )

            r_rdma.wait_recv()
            l_rdma.wait_recv()

        h = m_per // 2
        base = or_ref[N_STEPS] * m_per
        out_ref[pl.ds(base, h), :] = _gemm(rbuf[0, pl.ds(0, h)], w_ref[...])
        out_ref[pl.ds(base + h, h), :] = _gemm(lbuf[0, pl.ds(h, h)], w_ref[...])

    return pl.pallas_call(
        body,
        out_shape=jax.ShapeDtypeStruct((N_DEV * m_per, n_per), jnp.float32),
        in_specs=[
            pl.BlockSpec(memory_space=pltpu.SMEM),
            pl.BlockSpec(memory_space=pltpu.SMEM),
            pl.BlockSpec(memory_space=pltpu.SMEM),
            pl.BlockSpec(memory_space=pltpu.VMEM),
            pl.BlockSpec(memory_space=pltpu.VMEM),
        ],
        out_specs=pl.BlockSpec(memory_space=pltpu.VMEM),
        scratch_shapes=[
            pltpu.VMEM((2, m_per, k), jnp.float32),
            pltpu.VMEM((2, m_per, k), jnp.float32),
            pltpu.SemaphoreType.DMA((2,)),
            pltpu.SemaphoreType.DMA((2,)),
            pltpu.SemaphoreType.DMA((2,)),
            pltpu.SemaphoreType.DMA((2,)),
            pltpu.SemaphoreType.REGULAR,
            pltpu.SemaphoreType.REGULAR,
        ],
        compiler_params=pltpu.CompilerParams(collective_id=0),
    )(nbrs, origin_r, origin_l, x, w_mat)


# device time: 1438896 ns/iter; 2.0397x vs baseline; 1.0043x over previous
import jax
import jax.numpy as jnp
from jax import lax
from jax.experimental import pallas as pl
from jax.experimental.pallas import tpu as pltpu

N_DEV = 32
N_STEPS = 16

_P = [(0, 0), (1, 0), (2, 0), (3, 0), (3, 1), (2, 1), (1, 1), (0, 1),
      (0, 2), (1, 2), (2, 2), (3, 2), (3, 3), (2, 3), (1, 3), (0, 3)]
_CYCLE_COORDS = [(0, y, z) for y, z in _P] + [(1, y, z) for y, z in reversed(_P)]


def _logical(x, y, z):
    return z * 8 + y * 2 + (x if y % 2 == 0 else 1 - x)


_CYCLE = [_logical(*c) for c in _CYCLE_COORDS]
assert sorted(_CYCLE) == list(range(N_DEV))
_INV = [0] * N_DEV
for _i, _p in enumerate(_CYCLE):
    _INV[_p] = _i


def kernel(x, w_mat):
    m_per, k = x.shape
    _, n_per = w_mat.shape
    half = m_per // 2

    my_pos = lax.axis_index("i")
    cyc = jnp.asarray(_CYCLE, jnp.int32)
    inv = jnp.asarray(_INV, jnp.int32)
    cpos = inv[my_pos]
    steps = jnp.arange(N_STEPS + 1, dtype=jnp.int32)
    nbrs = jnp.stack([cyc[(cpos - 1) % N_DEV], cyc[(cpos + 1) % N_DEV]])
    origin_r = cyc[(cpos - steps) % N_DEV]
    origin_l = cyc[(cpos + steps[:N_STEPS]) % N_DEV]

    def _gemm(chunk, w):
        y = jnp.dot(chunk, w, preferred_element_type=jnp.float32)
        return jnp.maximum(y, 0.0)

    def body(nbr_ref, or_ref, ol_ref, x_ref, w_ref, out_ref,
             rbuf, lbuf, r_send, r_recv, l_send, l_recv, cred_r, cred_l):
        cl = nbr_ref[0]
        cr = nbr_ref[1]

        def _desc(buf, ssem, rsem, src_slot, dst_slot, j, dev):
            return pltpu.make_async_remote_copy(
                src_ref=buf.at[src_slot, pl.ds(j * half, half)],
                dst_ref=buf.at[dst_slot, pl.ds(j * half, half)],
                send_sem=ssem.at[src_slot, j],
                recv_sem=rsem.at[dst_slot, j],
                device_id=(dev,), device_id_type=pl.DeviceIdType.MESH,
            )

        barrier = pltpu.get_barrier_semaphore()
        for nbr in (cl, cr):
            pl.semaphore_signal(
                barrier, inc=1,
                device_id=(nbr,), device_id_type=pl.DeviceIdType.MESH,
            )
        pl.semaphore_wait(barrier, 2)

        rbuf[0] = x_ref[...]
        lbuf[0] = x_ref[...]

        for t in range(N_STEPS):
            s = t % 2
            r = (t + 1) % 2
            if t >= 1:
                pl.semaphore_wait(cred_r, 1)
                pl.semaphore_wait(cred_l, 1)

            in_flight = []
            for j in (0, 1):
                if t >= 1:
                    _desc(rbuf, r_send, r_recv, s, s, j, cr).wait_recv()
                if t <= N_STEPS - 2 or j == 0:
                    d = _desc(rbuf, r_send, r_recv, s, r, j, cr)
                    d.start()
                    in_flight.append(d)
                if t >= 1:
                    _desc(lbuf, l_send, l_recv, s, s, j, cl).wait_recv()
                if t <= N_STEPS - 2 or j == 1:
                    d = _desc(lbuf, l_send, l_recv, s, r, j, cl)
                    d.start()
                    in_flight.append(d)

            if t == 0:
                out_ref[pl.ds(or_ref[0] * m_per, m_per), :] = _gemm(
                    x_ref[...], w_ref[...])
            else:
                out_ref[pl.ds(or_ref[t] * m_per, m_per), :] = _gemm(
                    rbuf[s], w_ref[...])
                out_ref[pl.ds(ol_ref[t] * m_per, m_per), :] = _gemm(
                    lbuf[s], w_ref[...])

            for d in in_flight:
                d.wait_send()
            if t <= N_STEPS - 2:
                pl.semaphore_signal(
                    cred_r, inc=1,
                    device_id=(cl,), device_id_type=pl.DeviceIdType.MESH,
                )
                pl.semaphore_signal(
                    cred_l, inc=1,
                    device_id=(cr,), device_id_type=pl.DeviceIdType.MESH,
                )

        _desc(rbuf, r_send, r_recv, 0, 0, 0, cr).wait_recv()
        _desc(lbuf, l_send, l_recv, 0, 0, 1, cl).wait_recv()
        base = or_ref[N_STEPS] * m_per
        out_ref[pl.ds(base, half), :] = _gemm(
            rbuf[0, pl.ds(0, half)], w_ref[...])
        out_ref[pl.ds(base + half, half), :] = _gemm(
            lbuf[0, pl.ds(half, half)], w_ref[...])

    return pl.pallas_call(
        body,
        out_shape=jax.ShapeDtypeStruct((N_DEV * m_per, n_per), jnp.float32),
        in_specs=[
            pl.BlockSpec(memory_space=pltpu.SMEM),
            pl.BlockSpec(memory_space=pltpu.SMEM),
            pl.BlockSpec(memory_space=pltpu.SMEM),
            pl.BlockSpec(memory_space=pltpu.VMEM),
            pl.BlockSpec(memory_space=pltpu.VMEM),
        ],
        out_specs=pl.BlockSpec(memory_space=pltpu.VMEM),
        scratch_shapes=[
            pltpu.VMEM((2, m_per, k), jnp.float32),
            pltpu.VMEM((2, m_per, k), jnp.float32),
            pltpu.SemaphoreType.DMA((2, 2)),
            pltpu.SemaphoreType.DMA((2, 2)),
            pltpu.SemaphoreType.DMA((2, 2)),
            pltpu.SemaphoreType.DMA((2, 2)),
            pltpu.SemaphoreType.REGULAR,
            pltpu.SemaphoreType.REGULAR,
        ],
        compiler_params=pltpu.CompilerParams(collective_id=0),
    )(nbrs, origin_r, origin_l, x, w_mat)


# device time: 1425378 ns/iter; 2.0591x vs baseline; 1.0095x over previous
import jax
import jax.numpy as jnp
from jax import lax
from jax.experimental import pallas as pl
from jax.experimental.pallas import tpu as pltpu

N_DEV = 32
N_STEPS = 16

_P = [(0, 0), (1, 0), (2, 0), (3, 0), (3, 1), (2, 1), (1, 1), (0, 1),
      (0, 2), (1, 2), (2, 2), (3, 2), (3, 3), (2, 3), (1, 3), (0, 3)]
_CYCLE_COORDS = [(0, y, z) for y, z in _P] + [(1, y, z) for y, z in reversed(_P)]


def _logical(x, y, z):
    return z * 8 + y * 2 + (x if y % 2 == 0 else 1 - x)


_CYCLE = [_logical(*c) for c in _CYCLE_COORDS]
assert sorted(_CYCLE) == list(range(N_DEV))
_INV = [0] * N_DEV
for _i, _p in enumerate(_CYCLE):
    _INV[_p] = _i


def kernel(x, w_mat):
    m_per, k = x.shape
    _, n_per = w_mat.shape
    half = m_per // 2

    my_pos = lax.axis_index("i")
    cyc = jnp.asarray(_CYCLE, jnp.int32)
    inv = jnp.asarray(_INV, jnp.int32)
    cpos = inv[my_pos]
    steps = jnp.arange(N_STEPS + 1, dtype=jnp.int32)
    nbrs = jnp.stack([cyc[(cpos - 1) % N_DEV], cyc[(cpos + 1) % N_DEV]])
    origin_r = cyc[(cpos - steps) % N_DEV]
    origin_l = cyc[(cpos + steps[:N_STEPS]) % N_DEV]

    def _gemm(chunk, w):
        y = jnp.dot(chunk, w, preferred_element_type=jnp.float32)
        return jnp.maximum(y, 0.0)

    def body(nbr_ref, or_ref, ol_ref, x_ref, w_ref, out_ref,
             rbuf, lbuf, r_send, r_recv, l_send, l_recv,
             cred_r0, cred_r1, cred_l0, cred_l1):
        cl = nbr_ref[0]
        cr = nbr_ref[1]
        cred_r = (cred_r0, cred_r1)
        cred_l = (cred_l0, cred_l1)

        def _desc(buf, ssem, rsem, src_slot, dst_slot, j, dev):
            return pltpu.make_async_remote_copy(
                src_ref=buf.at[src_slot, pl.ds(j * half, half)],
                dst_ref=buf.at[dst_slot, pl.ds(j * half, half)],
                send_sem=ssem.at[src_slot, j],
                recv_sem=rsem.at[dst_slot, j],
                device_id=(dev,), device_id_type=pl.DeviceIdType.MESH,
            )

        barrier = pltpu.get_barrier_semaphore()
        for nbr in (cl, cr):
            pl.semaphore_signal(
                barrier, inc=1,
                device_id=(nbr,), device_id_type=pl.DeviceIdType.MESH,
            )
        pl.semaphore_wait(barrier, 2)

        def _src_desc(buf, ssem, rsem, t, r, j, dev):
            src = (x_ref.at[pl.ds(j * half, half)] if t == 0
                   else buf.at[t % 2, pl.ds(j * half, half)])
            return pltpu.make_async_remote_copy(
                src_ref=src,
                dst_ref=buf.at[r, pl.ds(j * half, half)],
                send_sem=ssem.at[t % 2, j],
                recv_sem=rsem.at[r, j],
                device_id=(dev,), device_id_type=pl.DeviceIdType.MESH,
            )

        for t in range(N_STEPS):
            s = t % 2
            r = (t + 1) % 2
            in_flight = {}
            for j in (0, 1):
                if t >= 1:
                    pl.semaphore_wait(cred_r[j], 1)
                    _desc(rbuf, r_send, r_recv, s, s, j, cr).wait_recv()
                if t <= N_STEPS - 2 or j == 0:
                    d = _src_desc(rbuf, r_send, r_recv, t, r, j, cr)
                    d.start()
                    in_flight[("r", j)] = d
                if t >= 1:
                    pl.semaphore_wait(cred_l[j], 1)
                    _desc(lbuf, l_send, l_recv, s, s, j, cl).wait_recv()
                if t <= N_STEPS - 2 or j == 1:
                    d = _src_desc(lbuf, l_send, l_recv, t, r, j, cl)
                    d.start()
                    in_flight[("l", j)] = d

            if t == 0:
                out_ref[pl.ds(or_ref[0] * m_per, m_per), :] = _gemm(
                    x_ref[...], w_ref[...])
            else:
                out_ref[pl.ds(or_ref[t] * m_per, m_per), :] = _gemm(
                    rbuf[s], w_ref[...])
                out_ref[pl.ds(ol_ref[t] * m_per, m_per), :] = _gemm(
                    lbuf[s], w_ref[...])

            for j in (0, 1):
                for key, sem, peer in ((("r", j), cred_r[j], cl),
                                       (("l", j), cred_l[j], cr)):
                    if key in in_flight:
                        in_flight[key].wait_send()
                        if t <= N_STEPS - 2:
                            pl.semaphore_signal(
                                sem, inc=1,
                                device_id=(peer,),
                                device_id_type=pl.DeviceIdType.MESH,
                            )

        _desc(rbuf, r_send, r_recv, 0, 0, 0, cr).wait_recv()
        _desc(lbuf, l_send, l_recv, 0, 0, 1, cl).wait_recv()
        base = or_ref[N_STEPS] * m_per
        out_ref[pl.ds(base, half), :] = _gemm(
            rbuf[0, pl.ds(0, half)], w_ref[...])
        out_ref[pl.ds(base + half, half), :] = _gemm(
            lbuf[0, pl.ds(half, half)], w_ref[...])

    return pl.pallas_call(
        body,
        out_shape=jax.ShapeDtypeStruct((N_DEV * m_per, n_per), jnp.float32),
        in_specs=[
            pl.BlockSpec(memory_space=pltpu.SMEM),
            pl.BlockSpec(memory_space=pltpu.SMEM),
            pl.BlockSpec(memory_space=pltpu.SMEM),
            pl.BlockSpec(memory_space=pltpu.VMEM),
            pl.BlockSpec(memory_space=pltpu.VMEM),
        ],
        out_specs=pl.BlockSpec(memory_space=pltpu.VMEM),
        scratch_shapes=[
            pltpu.VMEM((2, m_per, k), jnp.float32),
            pltpu.VMEM((2, m_per, k), jnp.float32),
            pltpu.SemaphoreType.DMA((2, 2)),
            pltpu.SemaphoreType.DMA((2, 2)),
            pltpu.SemaphoreType.DMA((2, 2)),
            pltpu.SemaphoreType.DMA((2, 2)),
            pltpu.SemaphoreType.REGULAR,
            pltpu.SemaphoreType.REGULAR,
            pltpu.SemaphoreType.REGULAR,
            pltpu.SemaphoreType.REGULAR,
        ],
        compiler_params=pltpu.CompilerParams(collective_id=0),
    )(nbrs, origin_r, origin_l, x, w_mat)
